# Initial kernel scaffold; baseline (speedup 1.0000x reference)
import jax
import jax.numpy as jnp
from jax.experimental import pallas as pl


def kernel(x_scalar, x_spherical, coord, batch_index, W_s1, b_s1, W_s2, b_s2, W_sph1, W_sph2):
    raise NotImplementedError("write your pallas kernel here")


# speedup vs baseline: 1.7311x; 1.7311x over previous
import functools
import math

import jax
import jax.numpy as jnp
from jax import lax
from jax.experimental import pallas as pl
from jax.experimental.pallas import tpu as pltpu
from jax.experimental.pallas import tpu_sc as plsc

N = 100000
NODE_DIM = 128
HID = 64
N_1E_IN = 64
N_1E_HID = 32
NUM_SEG = 512

NPAD = 102400
ROW_BLOCK = 800
GRID = NPAD // ROW_BLOCK
LAST_VALID_BLOCK = N // ROW_BLOCK - 1
NWORK = 32
W_PER = NPAD // NWORK
CHUNK = 128
NCHUNK = W_PER // CHUNK
SEG_D = 16


def _tc_body(xs_ref, xa_ref, xb_ref, w1_ref, b1_ref, w2_ref, b2_ref,
             wb1a_ref, wb1b_ref, msel_ref, mselt_ref, wb2_ref, out_ref):
    i = pl.program_id(0)
    hid = jnp.dot(xs_ref[...], w1_ref[...],
                  preferred_element_type=jnp.float32) + b1_ref[...]
    hid = hid * jax.nn.sigmoid(hid)
    s = jnp.dot(hid, w2_ref[...], preferred_element_type=jnp.float32) + b2_ref[...]
    h = (jnp.dot(xa_ref[...], wb1a_ref[...], preferred_element_type=jnp.float32)
         + jnp.dot(xb_ref[...][:, 0:64], wb1b_ref[...],
                   preferred_element_type=jnp.float32))
    nrm2 = jnp.dot(h * h, msel_ref[...], preferred_element_type=jnp.float32)
    gate = jax.nn.sigmoid(jnp.sqrt(nrm2 + 1e-12))
    hg = h * jnp.dot(gate, mselt_ref[...], preferred_element_type=jnp.float32)
    out4 = jnp.dot(hg, wb2_ref[...], preferred_element_type=jnp.float32)
    atom = out4 * s
    rows = i * ROW_BLOCK + lax.broadcasted_iota(jnp.int32, (ROW_BLOCK, SEG_D), 0)
    out_ref[...] = jnp.where(rows < N, atom, 0.0)


def _clamped(col):
    return lambda i: (jnp.minimum(i, LAST_VALID_BLOCK), col)


_tc_call = pl.pallas_call(
    _tc_body,
    grid=(GRID,),
    in_specs=[
        pl.BlockSpec((ROW_BLOCK, NODE_DIM), _clamped(0)),
        pl.BlockSpec((ROW_BLOCK, 128), _clamped(1)),
        pl.BlockSpec((ROW_BLOCK, 128), _clamped(2)),
        pl.BlockSpec((NODE_DIM, HID), lambda i: (0, 0)),
        pl.BlockSpec((1, HID), lambda i: (0, 0)),
        pl.BlockSpec((HID, 1), lambda i: (0, 0)),
        pl.BlockSpec((1, 1), lambda i: (0, 0)),
        pl.BlockSpec((128, 3 * N_1E_HID), lambda i: (0, 0)),
        pl.BlockSpec((N_1E_IN, 3 * N_1E_HID), lambda i: (0, 0)),
        pl.BlockSpec((3 * N_1E_HID, N_1E_HID), lambda i: (0, 0)),
        pl.BlockSpec((N_1E_HID, 3 * N_1E_HID), lambda i: (0, 0)),
        pl.BlockSpec((3 * N_1E_HID, SEG_D), lambda i: (0, 0)),
    ],
    out_specs=pl.BlockSpec((ROW_BLOCK, SEG_D), lambda i: (i, 0)),
    out_shape=jax.ShapeDtypeStruct((NPAD, SEG_D), jnp.float32),
    compiler_params=pltpu.CompilerParams(dimension_semantics=("arbitrary",)),
)


@functools.cache
def _make_sc_segsum():
    mesh = plsc.VectorSubcoreMesh(core_axis_name="c", subcore_axis_name="s")

    @functools.partial(
        pl.kernel,
        out_type=jax.ShapeDtypeStruct((2, NUM_SEG, SEG_D), jnp.float32),
        mesh=mesh,
        scratch_types=[
            pltpu.VMEM((NCHUNK, CHUNK), jnp.int32),
            pltpu.VMEM((W_PER, SEG_D), jnp.float32),
            pltpu.VMEM_SHARED((NUM_SEG, SEG_D), jnp.float32),
        ],
        compiler_params=pltpu.CompilerParams(use_tc_tiling_on_sc=False),
    )
    def _sc_segsum(atom_hbm, idx_hbm, zero_hbm, out_hbm, idx_v, vals_v, acc_sh):
        c = lax.axis_index("c")
        s = lax.axis_index("s")
        w = c * 16 + s

        @pl.when(s == 0)
        def _():
            pltpu.sync_copy(zero_hbm, acc_sh)

        pltpu.sync_copy(idx_hbm.at[w], idx_v)
        pltpu.sync_copy(atom_hbm.at[pl.ds(w * W_PER, W_PER)], vals_v)
        plsc.subcore_barrier()

        def body(j, carry):
            pltpu.sync_copy(vals_v.at[pl.ds(j * CHUNK, CHUNK)],
                            acc_sh.at[idx_v.at[j]], add=True)
            return carry

        lax.fori_loop(0, NCHUNK, body, 0)
        plsc.subcore_barrier()

        @pl.when(s == 0)
        def _():
            pltpu.sync_copy(acc_sh, out_hbm.at[c])

    return _sc_segsum


def kernel(x_scalar, x_spherical, coord, batch_index,
           W_s1, b_s1, W_s2, b_s2, W_sph1, W_sph2):
    del coord
    f32 = jnp.float32
    eye3 = jnp.eye(3, dtype=f32)
    wb1 = jnp.kron(W_sph1 / math.sqrt(float(N_1E_IN)), eye3)
    wb1a, wb1b = wb1[0:128], wb1[128:192]
    msel = jnp.kron(jnp.eye(N_1E_HID, dtype=f32), jnp.ones((3, 1), f32))
    mselt = msel.T
    perm = jnp.zeros((3, SEG_D), f32).at[0, 1].set(1.0).at[1, 2].set(1.0).at[2, 0].set(1.0)
    wb2 = jnp.kron(W_sph2 / math.sqrt(float(N_1E_HID)), perm)

    atom = _tc_call(
        x_scalar, x_spherical, x_spherical,
        W_s1, b_s1.reshape(1, HID), W_s2, b_s2.reshape(1, 1),
        wb1a, wb1b, msel, mselt, wb2,
    )

    bi = jnp.zeros((NPAD,), jnp.int32).at[:N].set(batch_index.astype(jnp.int32))
    bi = bi.reshape(NWORK, NCHUNK, CHUNK)
    zeros = jnp.zeros((NUM_SEG, SEG_D), f32)
    parts = _make_sc_segsum()(atom, bi, zeros)
    return (parts[0] + parts[1])[:, :3]


# speedup vs baseline: 1.9729x; 1.1397x over previous
import functools
import math

import jax
import jax.numpy as jnp
from jax import lax
from jax.experimental import pallas as pl
from jax.experimental.pallas import tpu as pltpu
from jax.experimental.pallas import tpu_sc as plsc

N = 100000
NODE_DIM = 128
HID = 64
N_1E_IN = 64
N_1E_HID = 32
NUM_SEG = 512

NPAD = 102400
ROW_BLOCK = 1600
GRID = NPAD // ROW_BLOCK
LAST_VALID_BLOCK = (N + ROW_BLOCK - 1) // ROW_BLOCK - 1
NWORK = 32
W_PER = NPAD // NWORK
CHUNK = 128
NCHUNK = W_PER // CHUNK
SEG_D = 16


def _tc_body(xs_ref, xa_ref, xb_ref, w1_ref, b1_ref, w2_ref, b2_ref,
             wb1a_ref, wb1b_ref, msel_ref, mselt_ref, wb2_ref, out_ref):
    i = pl.program_id(0)
    hid = jnp.dot(xs_ref[...], w1_ref[...],
                  preferred_element_type=jnp.float32) + b1_ref[...]
    hid = hid * jax.nn.sigmoid(hid)
    s = jnp.dot(hid, w2_ref[...], preferred_element_type=jnp.float32) + b2_ref[...]
    h = (jnp.dot(xa_ref[...], wb1a_ref[...], preferred_element_type=jnp.float32)
         + jnp.dot(xb_ref[...][:, 0:64], wb1b_ref[...],
                   preferred_element_type=jnp.float32))
    nrm2 = jnp.dot(h * h, msel_ref[...], preferred_element_type=jnp.float32)
    gate = jax.nn.sigmoid(jnp.sqrt(nrm2 + 1e-12))
    hg = h * jnp.dot(gate, mselt_ref[...], preferred_element_type=jnp.float32)
    out4 = jnp.dot(hg, wb2_ref[...], preferred_element_type=jnp.float32)
    atom = out4 * s
    rows = i * ROW_BLOCK + lax.broadcasted_iota(jnp.int32, (ROW_BLOCK, SEG_D), 0)
    out_ref[...] = jnp.where(rows < N, atom, 0.0)


def _clamped(col):
    return lambda i: (jnp.minimum(i, LAST_VALID_BLOCK), col)


_tc_call = pl.pallas_call(
    _tc_body,
    grid=(GRID,),
    in_specs=[
        pl.BlockSpec((ROW_BLOCK, NODE_DIM), _clamped(0)),
        pl.BlockSpec((ROW_BLOCK, 128), _clamped(1)),
        pl.BlockSpec((ROW_BLOCK, 128), _clamped(2)),
        pl.BlockSpec((NODE_DIM, HID), lambda i: (0, 0)),
        pl.BlockSpec((1, HID), lambda i: (0, 0)),
        pl.BlockSpec((HID, 1), lambda i: (0, 0)),
        pl.BlockSpec((1, 1), lambda i: (0, 0)),
        pl.BlockSpec((128, 3 * N_1E_HID), lambda i: (0, 0)),
        pl.BlockSpec((N_1E_IN, 3 * N_1E_HID), lambda i: (0, 0)),
        pl.BlockSpec((3 * N_1E_HID, N_1E_HID), lambda i: (0, 0)),
        pl.BlockSpec((N_1E_HID, 3 * N_1E_HID), lambda i: (0, 0)),
        pl.BlockSpec((3 * N_1E_HID, SEG_D), lambda i: (0, 0)),
    ],
    out_specs=pl.BlockSpec((ROW_BLOCK, SEG_D), lambda i: (i, 0)),
    out_shape=jax.ShapeDtypeStruct((NPAD, SEG_D), jnp.float32),
    compiler_params=pltpu.CompilerParams(dimension_semantics=("parallel",)),
)


@functools.cache
def _make_sc_segsum():
    mesh = plsc.VectorSubcoreMesh(core_axis_name="c", subcore_axis_name="s")

    @functools.partial(
        pl.kernel,
        out_type=jax.ShapeDtypeStruct((2, NUM_SEG, SEG_D), jnp.float32),
        mesh=mesh,
        scratch_types=[
            pltpu.VMEM((NCHUNK, CHUNK), jnp.int32),
            pltpu.VMEM((W_PER, SEG_D), jnp.float32),
            pltpu.VMEM_SHARED((NUM_SEG, SEG_D), jnp.float32),
        ],
        compiler_params=pltpu.CompilerParams(use_tc_tiling_on_sc=False),
    )
    def _sc_segsum(atom_hbm, idx_hbm, zero_hbm, out_hbm, idx_v, vals_v, acc_sh):
        c = lax.axis_index("c")
        s = lax.axis_index("s")
        w = c * 16 + s

        @pl.when(s == 0)
        def _():
            pltpu.sync_copy(zero_hbm, acc_sh)

        pltpu.sync_copy(idx_hbm.at[w], idx_v)
        pltpu.sync_copy(atom_hbm.at[pl.ds(w * W_PER, W_PER)], vals_v)
        plsc.subcore_barrier()

        def body(j, carry):
            pltpu.sync_copy(vals_v.at[pl.ds(j * CHUNK, CHUNK)],
                            acc_sh.at[idx_v.at[j]], add=True)
            return carry

        lax.fori_loop(0, NCHUNK, body, 0)
        plsc.subcore_barrier()

        @pl.when(s == 0)
        def _():
            pltpu.sync_copy(acc_sh, out_hbm.at[c])

    return _sc_segsum


def kernel(x_scalar, x_spherical, coord, batch_index,
           W_s1, b_s1, W_s2, b_s2, W_sph1, W_sph2):
    del coord
    f32 = jnp.float32
    eye3 = jnp.eye(3, dtype=f32)
    wb1 = jnp.kron(W_sph1 / math.sqrt(float(N_1E_IN)), eye3)
    wb1a, wb1b = wb1[0:128], wb1[128:192]
    msel = jnp.kron(jnp.eye(N_1E_HID, dtype=f32), jnp.ones((3, 1), f32))
    mselt = msel.T
    perm = jnp.zeros((3, SEG_D), f32).at[0, 1].set(1.0).at[1, 2].set(1.0).at[2, 0].set(1.0)
    wb2 = jnp.kron(W_sph2 / math.sqrt(float(N_1E_HID)), perm)

    atom = _tc_call(
        x_scalar, x_spherical, x_spherical,
        W_s1, b_s1.reshape(1, HID), W_s2, b_s2.reshape(1, 1),
        wb1a, wb1b, msel, mselt, wb2,
    )

    bi = jnp.zeros((NPAD,), jnp.int32).at[:N].set(batch_index.astype(jnp.int32))
    bi = bi.reshape(NWORK, NCHUNK, CHUNK)
    zeros = jnp.zeros((NUM_SEG, SEG_D), f32)
    parts = _make_sc_segsum()(atom, bi, zeros)
    return (parts[0] + parts[1])[:, :3]


# speedup vs baseline: 2.1093x; 1.0691x over previous
import functools
import math

import jax
import jax.numpy as jnp
from jax import lax
from jax.experimental import pallas as pl
from jax.experimental.pallas import tpu as pltpu
from jax.experimental.pallas import tpu_sc as plsc

N = 100000
NODE_DIM = 128
HID = 64
N_1E_IN = 64
N_1E_HID = 32
NUM_SEG = 512

NPAD = 102400
ROW_BLOCK = 3200
GRID = NPAD // ROW_BLOCK
LAST_VALID_BLOCK = (N + ROW_BLOCK - 1) // ROW_BLOCK - 1
NWORK = 32
W_PER = NPAD // NWORK
CHUNK = 128
NCHUNK = W_PER // CHUNK
SEG_D = 16


def _tc_body(xs_ref, xa_ref, xb_ref, w1_ref, b1_ref, w2_ref, b2_ref,
             wb1a_ref, wb1b_ref, msel_ref, mselt_ref, wb2_ref, out_ref):
    i = pl.program_id(0)
    hid = jnp.dot(xs_ref[...], w1_ref[...],
                  preferred_element_type=jnp.float32) + b1_ref[...]
    hid = hid * jax.nn.sigmoid(hid)
    s = jnp.dot(hid, w2_ref[...], preferred_element_type=jnp.float32) + b2_ref[...]
    h = (jnp.dot(xa_ref[...], wb1a_ref[...], preferred_element_type=jnp.float32)
         + jnp.dot(xb_ref[...][:, 0:64], wb1b_ref[...],
                   preferred_element_type=jnp.float32))
    nrm2 = jnp.dot(h * h, msel_ref[...], preferred_element_type=jnp.float32)
    gate = jax.nn.sigmoid(jnp.sqrt(nrm2 + 1e-12))
    hg = h * jnp.dot(gate, mselt_ref[...], preferred_element_type=jnp.float32)
    out4 = jnp.dot(hg, wb2_ref[...], preferred_element_type=jnp.float32)
    atom = out4 * s
    rows = i * ROW_BLOCK + lax.broadcasted_iota(jnp.int32, (ROW_BLOCK, SEG_D), 0)
    out_ref[...] = jnp.where(rows < N, atom, 0.0)


def _clamped(col):
    return lambda i: (jnp.minimum(i, LAST_VALID_BLOCK), col)


_tc_call = pl.pallas_call(
    _tc_body,
    grid=(GRID,),
    in_specs=[
        pl.BlockSpec((ROW_BLOCK, NODE_DIM), _clamped(0)),
        pl.BlockSpec((ROW_BLOCK, 128), _clamped(1)),
        pl.BlockSpec((ROW_BLOCK, 128), _clamped(2)),
        pl.BlockSpec((NODE_DIM, HID), lambda i: (0, 0)),
        pl.BlockSpec((1, HID), lambda i: (0, 0)),
        pl.BlockSpec((HID, 1), lambda i: (0, 0)),
        pl.BlockSpec((1, 1), lambda i: (0, 0)),
        pl.BlockSpec((128, 3 * N_1E_HID), lambda i: (0, 0)),
        pl.BlockSpec((N_1E_IN, 3 * N_1E_HID), lambda i: (0, 0)),
        pl.BlockSpec((3 * N_1E_HID, N_1E_HID), lambda i: (0, 0)),
        pl.BlockSpec((N_1E_HID, 3 * N_1E_HID), lambda i: (0, 0)),
        pl.BlockSpec((3 * N_1E_HID, SEG_D), lambda i: (0, 0)),
    ],
    out_specs=pl.BlockSpec((ROW_BLOCK, SEG_D), lambda i: (i, 0)),
    out_shape=jax.ShapeDtypeStruct((NPAD, SEG_D), jnp.float32),
    compiler_params=pltpu.CompilerParams(dimension_semantics=("parallel",)),
)


@functools.cache
def _make_sc_segsum():
    mesh = plsc.VectorSubcoreMesh(core_axis_name="c", subcore_axis_name="s")

    @functools.partial(
        pl.kernel,
        out_type=jax.ShapeDtypeStruct((2, NUM_SEG, SEG_D), jnp.float32),
        mesh=mesh,
        scratch_types=[
            pltpu.VMEM((NCHUNK, CHUNK), jnp.int32),
            pltpu.VMEM((W_PER, SEG_D), jnp.float32),
            pltpu.VMEM_SHARED((NUM_SEG, SEG_D), jnp.float32),
        ],
        compiler_params=pltpu.CompilerParams(use_tc_tiling_on_sc=False),
    )
    def _sc_segsum(atom_hbm, idx_hbm, zero_hbm, out_hbm, idx_v, vals_v, acc_sh):
        c = lax.axis_index("c")
        s = lax.axis_index("s")
        w = c * 16 + s

        @pl.when(s == 0)
        def _():
            pltpu.sync_copy(zero_hbm, acc_sh)

        pltpu.sync_copy(idx_hbm.at[w], idx_v)
        pltpu.sync_copy(atom_hbm.at[pl.ds(w * W_PER, W_PER)], vals_v)
        plsc.subcore_barrier()

        def body(j, carry):
            pltpu.sync_copy(vals_v.at[pl.ds(j * CHUNK, CHUNK)],
                            acc_sh.at[idx_v.at[j]], add=True)
            return carry

        lax.fori_loop(0, NCHUNK, body, 0)
        plsc.subcore_barrier()

        @pl.when(s == 0)
        def _():
            pltpu.sync_copy(acc_sh, out_hbm.at[c])

    return _sc_segsum


def kernel(x_scalar, x_spherical, coord, batch_index,
           W_s1, b_s1, W_s2, b_s2, W_sph1, W_sph2):
    del coord
    f32 = jnp.float32
    eye3 = jnp.eye(3, dtype=f32)
    wb1 = jnp.kron(W_sph1 / math.sqrt(float(N_1E_IN)), eye3)
    wb1a, wb1b = wb1[0:128], wb1[128:192]
    msel = jnp.kron(jnp.eye(N_1E_HID, dtype=f32), jnp.ones((3, 1), f32))
    mselt = msel.T
    perm = jnp.zeros((3, SEG_D), f32).at[0, 1].set(1.0).at[1, 2].set(1.0).at[2, 0].set(1.0)
    wb2 = jnp.kron(W_sph2 / math.sqrt(float(N_1E_HID)), perm)

    atom = _tc_call(
        x_scalar, x_spherical, x_spherical,
        W_s1, b_s1.reshape(1, HID), W_s2, b_s2.reshape(1, 1),
        wb1a, wb1b, msel, mselt, wb2,
    )

    bi = jnp.zeros((NPAD,), jnp.int32).at[:N].set(batch_index.astype(jnp.int32))
    bi = bi.reshape(NWORK, NCHUNK, CHUNK)
    zeros = jnp.zeros((NUM_SEG, SEG_D), f32)
    parts = _make_sc_segsum()(atom, bi, zeros)
    return (parts[0] + parts[1])[:, :3]


# speedup vs baseline: 2.1732x; 1.0303x over previous
import functools
import math

import jax
import jax.numpy as jnp
from jax import lax
from jax.experimental import pallas as pl
from jax.experimental.pallas import tpu as pltpu
from jax.experimental.pallas import tpu_sc as plsc

N = 100000
NODE_DIM = 128
HID = 64
N_1E_IN = 64
N_1E_HID = 32
NUM_SEG = 512

NPAD = 102400
ROW_BLOCK = 6400
GRID = NPAD // ROW_BLOCK
LAST_VALID_BLOCK = (N + ROW_BLOCK - 1) // ROW_BLOCK - 1
NWORK = 32
W_PER = NPAD // NWORK
CHUNK = 128
NCHUNK = W_PER // CHUNK
SEG_D = 16


def _tc_body(xs_ref, xa_ref, xb_ref, w1_ref, b1_ref, w2_ref, b2_ref,
             wb1a_ref, wb1b_ref, msel_ref, mselt_ref, wb2_ref, out_ref):
    i = pl.program_id(0)
    hid = jnp.dot(xs_ref[...], w1_ref[...],
                  preferred_element_type=jnp.float32) + b1_ref[...]
    hid = hid * jax.nn.sigmoid(hid)
    s = jnp.dot(hid, w2_ref[...], preferred_element_type=jnp.float32) + b2_ref[...]
    h = (jnp.dot(xa_ref[...], wb1a_ref[...], preferred_element_type=jnp.float32)
         + jnp.dot(xb_ref[...][:, 0:64], wb1b_ref[...],
                   preferred_element_type=jnp.float32))
    nrm2 = jnp.dot(h * h, msel_ref[...], preferred_element_type=jnp.float32)
    gate = jax.nn.sigmoid(jnp.sqrt(nrm2 + 1e-12))
    hg = h * jnp.dot(gate, mselt_ref[...], preferred_element_type=jnp.float32)
    out4 = jnp.dot(hg, wb2_ref[...], preferred_element_type=jnp.float32)
    atom = out4 * s
    rows = i * ROW_BLOCK + lax.broadcasted_iota(jnp.int32, (ROW_BLOCK, SEG_D), 0)
    out_ref[...] = jnp.where(rows < N, atom, 0.0)


def _clamped(col):
    return lambda i: (jnp.minimum(i, LAST_VALID_BLOCK), col)


_tc_call = pl.pallas_call(
    _tc_body,
    grid=(GRID,),
    in_specs=[
        pl.BlockSpec((ROW_BLOCK, NODE_DIM), _clamped(0)),
        pl.BlockSpec((ROW_BLOCK, 128), _clamped(1)),
        pl.BlockSpec((ROW_BLOCK, 128), _clamped(2)),
        pl.BlockSpec((NODE_DIM, HID), lambda i: (0, 0)),
        pl.BlockSpec((1, HID), lambda i: (0, 0)),
        pl.BlockSpec((HID, 1), lambda i: (0, 0)),
        pl.BlockSpec((1, 1), lambda i: (0, 0)),
        pl.BlockSpec((128, 3 * N_1E_HID), lambda i: (0, 0)),
        pl.BlockSpec((N_1E_IN, 3 * N_1E_HID), lambda i: (0, 0)),
        pl.BlockSpec((3 * N_1E_HID, N_1E_HID), lambda i: (0, 0)),
        pl.BlockSpec((N_1E_HID, 3 * N_1E_HID), lambda i: (0, 0)),
        pl.BlockSpec((3 * N_1E_HID, SEG_D), lambda i: (0, 0)),
    ],
    out_specs=pl.BlockSpec((ROW_BLOCK, SEG_D), lambda i: (i, 0)),
    out_shape=jax.ShapeDtypeStruct((NPAD, SEG_D), jnp.float32),
    compiler_params=pltpu.CompilerParams(dimension_semantics=("parallel",)),
)


@functools.cache
def _make_sc_segsum():
    mesh = plsc.VectorSubcoreMesh(core_axis_name="c", subcore_axis_name="s")

    @functools.partial(
        pl.kernel,
        out_type=jax.ShapeDtypeStruct((2, NUM_SEG, SEG_D), jnp.float32),
        mesh=mesh,
        scratch_types=[
            pltpu.VMEM((NCHUNK, CHUNK), jnp.int32),
            pltpu.VMEM((W_PER, SEG_D), jnp.float32),
            pltpu.VMEM_SHARED((NUM_SEG, SEG_D), jnp.float32),
        ],
        compiler_params=pltpu.CompilerParams(use_tc_tiling_on_sc=False),
    )
    def _sc_segsum(atom_hbm, idx_hbm, zero_hbm, out_hbm, idx_v, vals_v, acc_sh):
        c = lax.axis_index("c")
        s = lax.axis_index("s")
        w = c * 16 + s

        @pl.when(s == 0)
        def _():
            pltpu.sync_copy(zero_hbm, acc_sh)

        pltpu.sync_copy(idx_hbm.at[w], idx_v)
        pltpu.sync_copy(atom_hbm.at[pl.ds(w * W_PER, W_PER)], vals_v)
        plsc.subcore_barrier()

        def body(j, carry):
            pltpu.sync_copy(vals_v.at[pl.ds(j * CHUNK, CHUNK)],
                            acc_sh.at[idx_v.at[j]], add=True)
            return carry

        lax.fori_loop(0, NCHUNK, body, 0)
        plsc.subcore_barrier()

        @pl.when(s == 0)
        def _():
            pltpu.sync_copy(acc_sh, out_hbm.at[c])

    return _sc_segsum


def kernel(x_scalar, x_spherical, coord, batch_index,
           W_s1, b_s1, W_s2, b_s2, W_sph1, W_sph2):
    del coord
    f32 = jnp.float32
    eye3 = jnp.eye(3, dtype=f32)
    wb1 = jnp.kron(W_sph1 / math.sqrt(float(N_1E_IN)), eye3)
    wb1a, wb1b = wb1[0:128], wb1[128:192]
    msel = jnp.kron(jnp.eye(N_1E_HID, dtype=f32), jnp.ones((3, 1), f32))
    mselt = msel.T
    perm = jnp.zeros((3, SEG_D), f32).at[0, 1].set(1.0).at[1, 2].set(1.0).at[2, 0].set(1.0)
    wb2 = jnp.kron(W_sph2 / math.sqrt(float(N_1E_HID)), perm)

    atom = _tc_call(
        x_scalar, x_spherical, x_spherical,
        W_s1, b_s1.reshape(1, HID), W_s2, b_s2.reshape(1, 1),
        wb1a, wb1b, msel, mselt, wb2,
    )

    bi = jnp.zeros((NPAD,), jnp.int32).at[:N].set(batch_index.astype(jnp.int32))
    bi = bi.reshape(NWORK, NCHUNK, CHUNK)
    zeros = jnp.zeros((NUM_SEG, SEG_D), f32)
    parts = _make_sc_segsum()(atom, bi, zeros)
    return (parts[0] + parts[1])[:, :3]
